# Initial kernel scaffold; baseline (speedup 1.0000x reference)
import jax
import jax.numpy as jnp
from jax.experimental import pallas as pl


def kernel(encoder_output, durations):
    raise NotImplementedError("write your pallas kernel here")


# speedup vs baseline: 9.1851x; 9.1851x over previous
import jax
import jax.numpy as jnp
from jax import lax
from jax.experimental import pallas as pl
from jax.experimental.pallas import tpu as pltpu
from jax.experimental.pallas import tpu_sc as plsc

B, T, D = 16, 512, 384
F = 4 * T
L = 16
FRAMES = 1024
G = 128
NCHUNK = FRAMES // G
IPG = G // L


def _tile_body(enc_hbm, dur_hbm, out_hbm, dur_v, cnt_v, idx_v,
               buf0, buf1, gsem0, gsem1, wsem0, wsem1):
    wid = lax.axis_index("s") * 2 + lax.axis_index("c")
    b = wid // 2
    f0 = (wid % 2) * FRAMES
    i32 = jnp.int32

    pltpu.sync_copy(dur_hbm.at[b], dur_v)

    def _zero(m, c):
        cnt_v[pl.ds(m * L, L)] = jnp.zeros((L,), i32)
        return c
    lax.fori_loop(0, FRAMES // L, _zero, i32(0), unroll=8)

    one_v = jnp.ones((L,), i32)
    zero_v = jnp.zeros((L,), i32)

    def _scan(i, carry):
        run, base = carry
        v = jnp.maximum(dur_v[pl.ds(i * L, L)], 1)
        ends = plsc.cumsum(v) + run
        k = ends - f0
        plsc.addupdate_scatter(cnt_v, [k], one_v,
                               mask=(k >= 0) & (k < FRAMES))
        base = base + jnp.sum(jnp.where(k < 0, one_v, zero_v))
        return run + jnp.sum(v), base
    _, base = lax.fori_loop(0, T // L, _scan, (i32(0), i32(0)), unroll=2)

    row0 = base + b * T
    cap = b * T + (T - 1)

    def _psum(m, run):
        v = cnt_v[pl.ds(m * L, L)]
        s = plsc.cumsum(v) + (run + row0)
        idx_v[m // IPG, pl.ds((m % IPG) * L, L)] = jnp.minimum(s, cap)
        return run + jnp.sum(v)
    lax.fori_loop(0, FRAMES // L, _psum, i32(0), unroll=4)

    bufs = (buf0, buf1)
    gsems = (gsem0, gsem1)
    wsems = (wsem0, wsem1)

    pltpu.make_async_copy(enc_hbm.at[idx_v.at[0]], bufs[0], gsems[0]).start()
    for k in range(NCHUNK):
        s = k % 2
        pltpu.make_async_copy(enc_hbm.at[idx_v.at[k]], bufs[s],
                              gsems[s]).wait()
        if k + 1 < NCHUNK:
            if k >= 1:
                pltpu.make_async_copy(
                    bufs[1 - s], out_hbm.at[b, pl.ds(f0 + (k - 1) * G, G)],
                    wsems[1 - s]).wait()
            pltpu.make_async_copy(enc_hbm.at[idx_v.at[k + 1]], bufs[1 - s],
                                  gsems[1 - s]).start()
        pltpu.make_async_copy(bufs[s], out_hbm.at[b, pl.ds(f0 + k * G, G)],
                              wsems[s]).start()
    for k in (NCHUNK - 2, NCHUNK - 1):
        s = k % 2
        pltpu.make_async_copy(bufs[s], out_hbm.at[b, pl.ds(f0 + k * G, G)],
                              wsems[s]).wait()


@jax.jit
def kernel(encoder_output, durations):
    enc_flat = encoder_output.reshape(B * T, D)
    run = pl.kernel(
        _tile_body,
        out_type=jax.ShapeDtypeStruct((B, F, D), jnp.float32),
        mesh=plsc.VectorSubcoreMesh(core_axis_name="c", subcore_axis_name="s"),
        compiler_params=pltpu.CompilerParams(needs_layout_passes=False),
        scratch_types=[
            pltpu.VMEM((T,), jnp.int32),
            pltpu.VMEM((FRAMES,), jnp.int32),
            pltpu.VMEM((NCHUNK, G), jnp.int32),
            pltpu.VMEM((G, D), jnp.float32),
            pltpu.VMEM((G, D), jnp.float32),
            pltpu.SemaphoreType.DMA,
            pltpu.SemaphoreType.DMA,
            pltpu.SemaphoreType.DMA,
            pltpu.SemaphoreType.DMA,
        ],
    )
    return run(enc_flat, durations)


# speedup vs baseline: 9.5965x; 1.0448x over previous
import jax
import jax.numpy as jnp
from jax import lax
from jax.experimental import pallas as pl
from jax.experimental.pallas import tpu as pltpu
from jax.experimental.pallas import tpu_sc as plsc

B, T, D = 16, 512, 384
F = 4 * T
L = 16
FRAMES = 1024
G = 64
NCHUNK = FRAMES // G
IPG = G // L
NBUF = 4


def _tile_body(enc_hbm, dur_hbm, out_hbm, dur_v, cnt_v, idx_v,
               bufs, gsems, wsems):
    wid = lax.axis_index("s") * 2 + lax.axis_index("c")
    b = wid // 2
    f0 = (wid % 2) * FRAMES
    i32 = jnp.int32

    pltpu.sync_copy(dur_hbm.at[b], dur_v)

    for m in range(FRAMES // L):
        cnt_v[pl.ds(m * L, L)] = jnp.zeros((L,), i32)

    one_v = jnp.ones((L,), i32)
    zero_v = jnp.zeros((L,), i32)
    run = i32(0)
    base = i32(0)
    for i in range(T // L):
        v = jnp.maximum(dur_v[pl.ds(i * L, L)], 1)
        ends = plsc.cumsum(v) + run
        k = ends - f0
        plsc.addupdate_scatter(cnt_v, [k], one_v,
                               mask=(k >= 0) & (k < FRAMES))
        base = base + jnp.sum(jnp.where(k < 0, one_v, zero_v))
        run = run + jnp.sum(v)

    row0 = base + b * T
    cap = b * T + (T - 1)

    def _write(c):
        return pltpu.make_async_copy(
            bufs[c % NBUF], out_hbm.at[b, pl.ds(f0 + c * G, G)],
            wsems[c % NBUF])

    run = row0
    for c in range(NCHUNK):
        for m in range(IPG):
            v = cnt_v[pl.ds((c * IPG + m) * L, L)]
            s = plsc.cumsum(v) + run
            idx_v[c, pl.ds(m * L, L)] = jnp.minimum(s, cap)
            run = run + jnp.sum(v)
        if c >= NBUF:
            _write(c - NBUF).wait()
        pltpu.make_async_copy(enc_hbm.at[idx_v.at[c]], bufs[c % NBUF],
                              gsems[c % NBUF]).start()
        if c >= 1:
            pltpu.make_async_copy(enc_hbm.at[idx_v.at[c - 1]],
                                  bufs[(c - 1) % NBUF],
                                  gsems[(c - 1) % NBUF]).wait()
            _write(c - 1).start()
    c = NCHUNK - 1
    pltpu.make_async_copy(enc_hbm.at[idx_v.at[c]], bufs[c % NBUF],
                          gsems[c % NBUF]).wait()
    _write(c).start()
    for c in range(NCHUNK - NBUF, NCHUNK):
        _write(c).wait()


@jax.jit
def kernel(encoder_output, durations):
    enc_flat = encoder_output.reshape(B * T, D)
    run = pl.kernel(
        _tile_body,
        out_type=jax.ShapeDtypeStruct((B, F, D), jnp.float32),
        mesh=plsc.VectorSubcoreMesh(core_axis_name="c", subcore_axis_name="s"),
        compiler_params=pltpu.CompilerParams(needs_layout_passes=False),
        scratch_types=[
            pltpu.VMEM((T,), jnp.int32),
            pltpu.VMEM((FRAMES,), jnp.int32),
            pltpu.VMEM((NCHUNK, G), jnp.int32),
            [pltpu.VMEM((G, D), jnp.float32) for _ in range(NBUF)],
            [pltpu.SemaphoreType.DMA for _ in range(NBUF)],
            [pltpu.SemaphoreType.DMA for _ in range(NBUF)],
        ],
    )
    return run(enc_flat, durations)
